# baseline (speedup 1.0000x reference)
import functools

import jax
import jax.numpy as jnp
from jax.experimental import pallas as pl
from jax.experimental.pallas import tpu as pltpu
from jax.experimental.pallas import tpu_sc as plsc

N = 10000
E = 320000
D = 128
H = 128

NC = 2
NS = 16
NW = NC * NS
EP = E // NW
K = 80
NCHUNK = EP // K
NG = 5
GC = NCHUNK // NG
CW = 128
NP = 10240
RPT = NP // NS

_mesh = plsc.VectorSubcoreMesh(core_axis_name="c", subcore_axis_name="s")


@functools.partial(
    pl.kernel,
    out_type=jax.ShapeDtypeStruct((NC, NP, D), jnp.float32),
    mesh=_mesh,
    scratch_types=[
        pltpu.VMEM((GC, K), jnp.int32),
        pltpu.VMEM((GC, K), jnp.int32),
        pltpu.VMEM((K, D), jnp.float32),
        pltpu.VMEM((K, D), jnp.float32),
        pltpu.VMEM_SHARED((NP, D), jnp.float32),
        pltpu.SemaphoreType.DMA,
        pltpu.SemaphoreType.DMA,
        pltpu.SemaphoreType.DMA,
        pltpu.SemaphoreType.DMA,
    ],
)
def _sc_agg(x_hbm, src_hbm, dst_hbm, z_hbm, p_hbm,
            sidx_all, didx_all, rows0, rows1, acc, g0, g1, s0, s1):
    cid = jax.lax.axis_index("c")
    sid = jax.lax.axis_index("s")
    wid = cid * NS + sid

    pltpu.sync_copy(z_hbm, rows0)

    @pl.loop(0, RPT // K)
    def _(r):
        pltpu.sync_copy(rows0, acc.at[pl.ds(sid * RPT + r * K, K)])

    plsc.subcore_barrier()

    def _wait(sem, buf):
        pltpu.make_async_copy(z_hbm, buf, sem).wait()

    @pl.loop(0, NG)
    def _(g):
        pltpu.sync_copy(src_hbm.at[wid, g], sidx_all)
        pltpu.sync_copy(dst_hbm.at[wid, g], didx_all)
        pltpu.async_copy(x_hbm.at[sidx_all.at[0]], rows0, g0)
        pltpu.async_copy(x_hbm.at[sidx_all.at[1]], rows1, g1)

        @pl.loop(0, GC // 2 - 1)
        def _(r):
            j0 = 2 * r
            _wait(g0, rows0)
            pltpu.async_copy(rows0, acc.at[didx_all.at[j0]], s0, add=True)
            _wait(g1, rows1)
            pltpu.async_copy(rows1, acc.at[didx_all.at[j0 + 1]], s1, add=True)
            _wait(s0, rows0)
            pltpu.async_copy(x_hbm.at[sidx_all.at[j0 + 2]], rows0, g0)
            _wait(s1, rows1)
            pltpu.async_copy(x_hbm.at[sidx_all.at[j0 + 3]], rows1, g1)

        _wait(g0, rows0)
        pltpu.async_copy(rows0, acc.at[didx_all.at[GC - 3]], s0, add=True)
        _wait(g1, rows1)
        pltpu.async_copy(rows1, acc.at[didx_all.at[GC - 2]], s1, add=True)
        _wait(s0, rows0)
        pltpu.async_copy(x_hbm.at[sidx_all.at[GC - 1]], rows0, g0)
        _wait(g0, rows0)
        pltpu.sync_copy(rows0, acc.at[didx_all.at[GC - 1]], add=True)
        _wait(s1, rows1)

    plsc.subcore_barrier()

    @pl.loop(0, RPT // K)
    def _(r):
        rbase = sid * RPT + r * K
        pltpu.sync_copy(acc.at[pl.ds(rbase, K)], rows0)
        pltpu.sync_copy(rows0, p_hbm.at[cid, pl.ds(rbase, K)])


@functools.partial(
    pl.kernel,
    out_type=jax.ShapeDtypeStruct((NC, NP, CW), jnp.float32),
    mesh=_mesh,
    scratch_types=[
        pltpu.VMEM((GC, K), jnp.int32),
        pltpu.VMEM((K, CW), jnp.float32),
        pltpu.VMEM((K, CW), jnp.float32),
        pltpu.VMEM_SHARED((NP, CW), jnp.float32),
        pltpu.SemaphoreType.DMA,
        pltpu.SemaphoreType.DMA,
    ],
)
def _sc_count(dst_hbm, z16_hbm, ones_hbm, cnt_hbm,
              didx_all, z16buf, onesbuf, cntacc, s0, s1):
    cid = jax.lax.axis_index("c")
    sid = jax.lax.axis_index("s")
    wid = cid * NS + sid

    pltpu.sync_copy(z16_hbm, z16buf)
    pltpu.sync_copy(ones_hbm, onesbuf)

    @pl.loop(0, RPT // K)
    def _(r):
        pltpu.sync_copy(z16buf, cntacc.at[pl.ds(sid * RPT + r * K, K)])

    plsc.subcore_barrier()

    def _wait(sem):
        pltpu.make_async_copy(z16_hbm, z16buf, sem).wait()

    @pl.loop(0, NG)
    def _(g):
        pltpu.sync_copy(dst_hbm.at[wid, g], didx_all)
        pltpu.async_copy(onesbuf, cntacc.at[didx_all.at[0]], s0, add=True)
        pltpu.async_copy(onesbuf, cntacc.at[didx_all.at[1]], s1, add=True)

        @pl.loop(0, GC // 2 - 1)
        def _(r):
            j0 = 2 * r
            _wait(s0)
            pltpu.async_copy(onesbuf, cntacc.at[didx_all.at[j0 + 2]], s0, add=True)
            _wait(s1)
            pltpu.async_copy(onesbuf, cntacc.at[didx_all.at[j0 + 3]], s1, add=True)

        _wait(s0)
        pltpu.async_copy(onesbuf, cntacc.at[didx_all.at[GC - 1]], s0, add=True)
        _wait(s1)
        _wait(s0)

    plsc.subcore_barrier()

    @pl.loop(0, RPT // K)
    def _(r):
        rbase = sid * RPT + r * K
        pltpu.sync_copy(cntacc.at[pl.ds(rbase, K)], z16buf)
        pltpu.sync_copy(z16buf, cnt_hbm.at[cid, pl.ds(rbase, K)])


BN = 1000


def _tc_body(p_ref, cnt_ref, x_ref, wrel_ref, b_ref, wroot_ref, o_ref):
    psum = p_ref[0] + p_ref[1]
    c = (cnt_ref[0] + cnt_ref[1])[:, 0:1]
    agg = psum * (1.0 / jnp.maximum(c, 1.0))
    h = jax.lax.dot_general(agg, wrel_ref[...],
                            (((1,), (1,)), ((), ())),
                            preferred_element_type=jnp.float32)
    h = h + jax.lax.dot_general(x_ref[...], wroot_ref[...],
                                (((1,), (1,)), ((), ())),
                                preferred_element_type=jnp.float32)
    h = h + b_ref[...]
    o_ref[...] = jnp.where(h > 0.0, h, jnp.exp(h) - 1.0)


def _tc_epilogue(p, cnt, x, w_rel, b, w_root):
    return pl.pallas_call(
        _tc_body,
        grid=(N // BN,),
        in_specs=[
            pl.BlockSpec((NC, BN, D), lambda i: (0, i, 0)),
            pl.BlockSpec((NC, BN, CW), lambda i: (0, i, 0)),
            pl.BlockSpec((BN, D), lambda i: (i, 0)),
            pl.BlockSpec((H, D), lambda i: (0, 0)),
            pl.BlockSpec((1, H), lambda i: (0, 0)),
            pl.BlockSpec((H, D), lambda i: (0, 0)),
        ],
        out_specs=pl.BlockSpec((BN, H), lambda i: (i, 0)),
        out_shape=jax.ShapeDtypeStruct((N, H), jnp.float32),
    )(p, cnt, x, w_rel, b, w_root)


def kernel(node_embedding, edge_index, W_rel1, b1, W_root1, W_rel2, b2, W_root2):
    x = node_embedding
    src = edge_index[0]
    dst = edge_index[1]
    z = jnp.zeros((K, D), jnp.float32)
    z16 = jnp.zeros((K, CW), jnp.float32)
    ones = jnp.ones((K, CW), jnp.float32)

    src3 = src.reshape(NW, NG, GC, K)
    dst3 = dst.reshape(NW, NG, GC, K)

    cnt = _sc_count(dst3, z16, ones)
    p1 = _sc_agg(x, src3, dst3, z)
    x1 = _tc_epilogue(p1, cnt, x, W_rel1, b1.reshape(1, H), W_root1)
    p2 = _sc_agg(x1, src3, dst3, z)
    x2 = _tc_epilogue(p2, cnt, x1, W_rel2, b2.reshape(1, H), W_root2)
    return x2


# speedup vs baseline: 1.1671x; 1.1671x over previous
import functools

import jax
import jax.numpy as jnp
from jax.experimental import pallas as pl
from jax.experimental.pallas import tpu as pltpu
from jax.experimental.pallas import tpu_sc as plsc

N = 10000
E = 320000
D = 128
H = 128

NC = 2
NS = 16
NW = NC * NS
EP = E // NW
K = 80
NCHUNK = EP // K
NG = 5
GC = NCHUNK // NG
CW = 128
NP = 10240
RPT = NP // NS

_mesh = plsc.VectorSubcoreMesh(core_axis_name="c", subcore_axis_name="s")


@functools.partial(
    pl.kernel,
    out_type=jax.ShapeDtypeStruct((NC, NP, D), jnp.float32),
    mesh=_mesh,
    scratch_types=[
        pltpu.VMEM((GC, K), jnp.int32),
        pltpu.VMEM((GC, K), jnp.int32),
        pltpu.VMEM((K, D), jnp.float32),
        pltpu.VMEM((K, D), jnp.float32),
        pltpu.VMEM_SHARED((NP, D), jnp.float32),
        pltpu.SemaphoreType.DMA,
        pltpu.SemaphoreType.DMA,
        pltpu.SemaphoreType.DMA,
        pltpu.SemaphoreType.DMA,
    ],
)
def _sc_agg(x_hbm, src_hbm, dst_hbm, z_hbm, p_hbm,
            sidx_all, didx_all, rows0, rows1, acc, g0, g1, s0, s1):
    cid = jax.lax.axis_index("c")
    sid = jax.lax.axis_index("s")
    wid = cid * NS + sid

    pltpu.sync_copy(z_hbm, rows0)

    @pl.loop(0, RPT // K)
    def _(r):
        pltpu.sync_copy(rows0, acc.at[pl.ds(sid * RPT + r * K, K)])

    plsc.subcore_barrier()

    @pl.loop(0, NG)
    def _(g):
        pltpu.sync_copy(src_hbm.at[wid, g], sidx_all)
        pltpu.sync_copy(dst_hbm.at[wid, g], didx_all)
        pltpu.async_copy(x_hbm.at[sidx_all.at[0]], rows0, g0)

        @pl.loop(0, (GC - 1) // 2)
        def _(r):
            j0 = 2 * r
            pltpu.async_copy(x_hbm.at[sidx_all.at[j0 + 1]], rows1, g1)
            pltpu.make_async_copy(z_hbm, rows0, g0).wait()
            pltpu.sync_copy(rows0, acc.at[didx_all.at[j0]], add=True)
            pltpu.async_copy(x_hbm.at[sidx_all.at[j0 + 2]], rows0, g0)
            pltpu.make_async_copy(z_hbm, rows1, g1).wait()
            pltpu.sync_copy(rows1, acc.at[didx_all.at[j0 + 1]], add=True)

        pltpu.make_async_copy(z_hbm, rows0, g0).wait()
        pltpu.sync_copy(rows0, acc.at[didx_all.at[GC - 1]], add=True)

    plsc.subcore_barrier()

    @pl.loop(0, RPT // K)
    def _(r):
        rbase = sid * RPT + r * K
        pltpu.sync_copy(acc.at[pl.ds(rbase, K)], rows0)
        pltpu.sync_copy(rows0, p_hbm.at[cid, pl.ds(rbase, K)])


@functools.partial(
    pl.kernel,
    out_type=jax.ShapeDtypeStruct((NC, NP, CW), jnp.float32),
    mesh=_mesh,
    scratch_types=[
        pltpu.VMEM((GC, K), jnp.int32),
        pltpu.VMEM((K, CW), jnp.float32),
        pltpu.VMEM((K, CW), jnp.float32),
        pltpu.VMEM_SHARED((NP, CW), jnp.float32),
        pltpu.SemaphoreType.DMA,
        pltpu.SemaphoreType.DMA,
    ],
)
def _sc_count(dst_hbm, z16_hbm, ones_hbm, cnt_hbm,
              didx_all, z16buf, onesbuf, cntacc, s0, s1):
    cid = jax.lax.axis_index("c")
    sid = jax.lax.axis_index("s")
    wid = cid * NS + sid

    pltpu.sync_copy(z16_hbm, z16buf)
    pltpu.sync_copy(ones_hbm, onesbuf)

    @pl.loop(0, RPT // K)
    def _(r):
        pltpu.sync_copy(z16buf, cntacc.at[pl.ds(sid * RPT + r * K, K)])

    plsc.subcore_barrier()

    def _wait(sem):
        pltpu.make_async_copy(z16_hbm, z16buf, sem).wait()

    @pl.loop(0, NG)
    def _(g):
        pltpu.sync_copy(dst_hbm.at[wid, g], didx_all)
        pltpu.async_copy(onesbuf, cntacc.at[didx_all.at[0]], s0, add=True)
        pltpu.async_copy(onesbuf, cntacc.at[didx_all.at[1]], s1, add=True)

        @pl.loop(0, GC // 2 - 1)
        def _(r):
            j0 = 2 * r
            _wait(s0)
            pltpu.async_copy(onesbuf, cntacc.at[didx_all.at[j0 + 2]], s0, add=True)
            _wait(s1)
            pltpu.async_copy(onesbuf, cntacc.at[didx_all.at[j0 + 3]], s1, add=True)

        _wait(s0)
        pltpu.async_copy(onesbuf, cntacc.at[didx_all.at[GC - 1]], s0, add=True)
        _wait(s1)
        _wait(s0)

    plsc.subcore_barrier()

    @pl.loop(0, RPT // K)
    def _(r):
        rbase = sid * RPT + r * K
        pltpu.sync_copy(cntacc.at[pl.ds(rbase, K)], z16buf)
        pltpu.sync_copy(z16buf, cnt_hbm.at[cid, pl.ds(rbase, K)])


BN = 1000


def _tc_body(p_ref, cnt_ref, x_ref, wrel_ref, b_ref, wroot_ref, o_ref):
    psum = p_ref[0] + p_ref[1]
    c = (cnt_ref[0] + cnt_ref[1])[:, 0:1]
    agg = psum * (1.0 / jnp.maximum(c, 1.0))
    h = jax.lax.dot_general(agg, wrel_ref[...],
                            (((1,), (1,)), ((), ())),
                            preferred_element_type=jnp.float32)
    h = h + jax.lax.dot_general(x_ref[...], wroot_ref[...],
                                (((1,), (1,)), ((), ())),
                                preferred_element_type=jnp.float32)
    h = h + b_ref[...]
    o_ref[...] = jnp.where(h > 0.0, h, jnp.exp(h) - 1.0)


def _tc_epilogue(p, cnt, x, w_rel, b, w_root):
    return pl.pallas_call(
        _tc_body,
        grid=(N // BN,),
        in_specs=[
            pl.BlockSpec((NC, BN, D), lambda i: (0, i, 0)),
            pl.BlockSpec((NC, BN, CW), lambda i: (0, i, 0)),
            pl.BlockSpec((BN, D), lambda i: (i, 0)),
            pl.BlockSpec((H, D), lambda i: (0, 0)),
            pl.BlockSpec((1, H), lambda i: (0, 0)),
            pl.BlockSpec((H, D), lambda i: (0, 0)),
        ],
        out_specs=pl.BlockSpec((BN, H), lambda i: (i, 0)),
        out_shape=jax.ShapeDtypeStruct((N, H), jnp.float32),
    )(p, cnt, x, w_rel, b, w_root)


def kernel(node_embedding, edge_index, W_rel1, b1, W_root1, W_rel2, b2, W_root2):
    x = node_embedding
    src = edge_index[0]
    dst = edge_index[1]
    z = jnp.zeros((K, D), jnp.float32)
    z16 = jnp.zeros((K, CW), jnp.float32)
    ones = jnp.ones((K, CW), jnp.float32)

    src3 = src.reshape(NW, NG, GC, K)
    dst3 = dst.reshape(NW, NG, GC, K)

    cnt = _sc_count(dst3, z16, ones)
    p1 = _sc_agg(x, src3, dst3, z)
    x1 = _tc_epilogue(p1, cnt, x, W_rel1, b1.reshape(1, H), W_root1)
    p2 = _sc_agg(x1, src3, dst3, z)
    x2 = _tc_epilogue(p2, cnt, x1, W_rel2, b2.reshape(1, H), W_root2)
    return x2


# speedup vs baseline: 1.4238x; 1.2199x over previous
import dataclasses
import functools

import jax
import jax.numpy as jnp
from jax.experimental import pallas as pl
from jax.experimental.pallas import tpu as pltpu
from jax.experimental.pallas import tpu_sc as plsc

N = 10000
E = 320000
D = 128
H = 128

NC = 2
NS = 16
NW = NC * NS
EP = E // NW
K = 80
NCHUNK = EP // K
NG = 5
GC = NCHUNK // NG
CW = 128
NP = 10240
RPT = NP // NS

_mesh = plsc.VectorSubcoreMesh(core_axis_name="c", subcore_axis_name="s")


def _make_sc_agg(with_counts):
    if with_counts:
        out_type = (jax.ShapeDtypeStruct((NC, NP, D), jnp.float32),
                    jax.ShapeDtypeStruct((NW, 1, N), jnp.float32))
    else:
        out_type = jax.ShapeDtypeStruct((NC, NP, D), jnp.float32)
    scratch = [
        pltpu.VMEM((GC, K), jnp.int32),
        pltpu.VMEM((GC, K), jnp.int32),
        pltpu.VMEM((K, D), jnp.float32),
        pltpu.VMEM((K, D), jnp.float32),
        pltpu.VMEM_SHARED((NP, D), jnp.float32),
        pltpu.SemaphoreType.DMA,
        pltpu.SemaphoreType.DMA,
    ]
    if with_counts:
        scratch.append(pltpu.VMEM((1, N), jnp.float32))

    def body(*refs):
        if with_counts:
            (x_hbm, src_hbm, dst_hbm, z_hbm, p_hbm, cnt_hbm,
             sidx_all, didx_all, rows0, rows1, acc, g0, g1, cnt_local) = refs
        else:
            (x_hbm, src_hbm, dst_hbm, z_hbm, p_hbm,
             sidx_all, didx_all, rows0, rows1, acc, g0, g1) = refs

        cid = jax.lax.axis_index("c")
        sid = jax.lax.axis_index("s")
        wid = cid * NS + sid

        pltpu.sync_copy(z_hbm, rows0)

        @pl.loop(0, RPT // K)
        def _(r):
            pltpu.sync_copy(rows0, acc.at[pl.ds(sid * RPT + r * K, K)])

        if with_counts:
            zero16 = jnp.zeros((16,), jnp.float32)

            @pl.loop(0, N // 16)
            def _(i):
                cnt_local[0, pl.ds(i * 16, 16)] = zero16

            izero = jnp.zeros((16,), jnp.int32)
            fone = jnp.full((16,), 1.0, jnp.float32)

            def count_chunk(j):
                @pl.loop(0, K // 16)
                def _(m):
                    dvec = didx_all[j, pl.ds(m * 16, 16)]
                    plsc.addupdate_scatter(cnt_local, [izero, dvec], fone)
        else:
            def count_chunk(j):
                pass

        plsc.subcore_barrier()

        @pl.loop(0, NG)
        def _(g):
            pltpu.sync_copy(src_hbm.at[wid, g], sidx_all)
            pltpu.sync_copy(dst_hbm.at[wid, g], didx_all)
            pltpu.async_copy(x_hbm.at[sidx_all.at[0]], rows0, g0)

            @pl.loop(0, (GC - 1) // 2)
            def _(r):
                j0 = 2 * r
                pltpu.async_copy(x_hbm.at[sidx_all.at[j0 + 1]], rows1, g1)
                pltpu.make_async_copy(z_hbm, rows0, g0).wait()
                pltpu.sync_copy(rows0, acc.at[didx_all.at[j0]], add=True)
                pltpu.async_copy(x_hbm.at[sidx_all.at[j0 + 2]], rows0, g0)
                count_chunk(j0)
                pltpu.make_async_copy(z_hbm, rows1, g1).wait()
                pltpu.sync_copy(rows1, acc.at[didx_all.at[j0 + 1]], add=True)
                count_chunk(j0 + 1)

            pltpu.make_async_copy(z_hbm, rows0, g0).wait()
            pltpu.sync_copy(rows0, acc.at[didx_all.at[GC - 1]], add=True)
            count_chunk(GC - 1)

        plsc.subcore_barrier()

        @pl.loop(0, RPT // K)
        def _(r):
            rbase = sid * RPT + r * K
            pltpu.sync_copy(acc.at[pl.ds(rbase, K)], rows0)
            pltpu.sync_copy(rows0, p_hbm.at[cid, pl.ds(rbase, K)])

        if with_counts:
            pltpu.sync_copy(cnt_local, cnt_hbm.at[wid])

    cp = pltpu.CompilerParams()
    if with_counts and "needs_layout_passes" in pltpu.CompilerParams.__dataclass_fields__:
        cp = dataclasses.replace(cp, needs_layout_passes=False)
    return functools.partial(
        pl.kernel, out_type=out_type, mesh=_mesh, scratch_types=scratch,
        compiler_params=cp)(body)


_sc_agg_counts = _make_sc_agg(True)
_sc_agg = _make_sc_agg(False)


BN = 1000


def _tc_body(p_ref, cnt_ref, x_ref, wrel_ref, b_ref, wroot_ref, o_ref):
    psum = p_ref[0] + p_ref[1]
    c = jnp.sum(cnt_ref[...], axis=1, keepdims=True)
    agg = psum * (1.0 / jnp.maximum(c, 1.0))
    h = jax.lax.dot_general(agg, wrel_ref[...],
                            (((1,), (1,)), ((), ())),
                            preferred_element_type=jnp.float32)
    h = h + jax.lax.dot_general(x_ref[...], wroot_ref[...],
                                (((1,), (1,)), ((), ())),
                                preferred_element_type=jnp.float32)
    h = h + b_ref[...]
    o_ref[...] = jnp.where(h > 0.0, h, jnp.exp(h) - 1.0)


def _tc_epilogue(p, cnt, x, w_rel, b, w_root):
    return pl.pallas_call(
        _tc_body,
        grid=(N // BN,),
        in_specs=[
            pl.BlockSpec((NC, BN, D), lambda i: (0, i, 0)),
            pl.BlockSpec((BN, NW), lambda i: (i, 0)),
            pl.BlockSpec((BN, D), lambda i: (i, 0)),
            pl.BlockSpec((H, D), lambda i: (0, 0)),
            pl.BlockSpec((1, H), lambda i: (0, 0)),
            pl.BlockSpec((H, D), lambda i: (0, 0)),
        ],
        out_specs=pl.BlockSpec((BN, H), lambda i: (i, 0)),
        out_shape=jax.ShapeDtypeStruct((N, H), jnp.float32),
    )(p, cnt, x, w_rel, b, w_root)


def kernel(node_embedding, edge_index, W_rel1, b1, W_root1, W_rel2, b2, W_root2):
    x = node_embedding
    src = edge_index[0]
    dst = edge_index[1]
    z = jnp.zeros((K, D), jnp.float32)

    src3 = src.reshape(NW, NG, GC, K)
    dst3 = dst.reshape(NW, NG, GC, K)

    p1, cnt32 = _sc_agg_counts(x, src3, dst3, z)
    cnt = cnt32.reshape(NW, N).T
    x1 = _tc_epilogue(p1, cnt, x, W_rel1, b1.reshape(1, H), W_root1)
    p2 = _sc_agg(x1, src3, dst3, z)
    x2 = _tc_epilogue(p2, cnt, x1, W_rel2, b2.reshape(1, H), W_root2)
    return x2


# speedup vs baseline: 1.4455x; 1.0152x over previous
import dataclasses
import functools

import jax
import jax.numpy as jnp
from jax.experimental import pallas as pl
from jax.experimental.pallas import tpu as pltpu
from jax.experimental.pallas import tpu_sc as plsc

N = 10000
E = 320000
D = 128
H = 128

NC = 2
NS = 16
NW = NC * NS
EP = E // NW
K = 80
NCHUNK = EP // K
NG = 5
GC = NCHUNK // NG
CW = 128
NP = 10240
RPT = NP // NS

_mesh = plsc.VectorSubcoreMesh(core_axis_name="c", subcore_axis_name="s")


def _make_sc_agg(with_counts):
    if with_counts:
        out_type = (jax.ShapeDtypeStruct((NC, NP, D), jnp.float32),
                    jax.ShapeDtypeStruct((NW, 1, N), jnp.float32))
    else:
        out_type = jax.ShapeDtypeStruct((NC, NP, D), jnp.float32)
    scratch = [
        pltpu.VMEM((GC, K), jnp.int32),
        pltpu.VMEM((GC, K), jnp.int32),
        pltpu.VMEM((K, D), jnp.float32),
        pltpu.VMEM((K, D), jnp.float32),
        pltpu.VMEM_SHARED((NP, D), jnp.float32),
        pltpu.SemaphoreType.DMA,
        pltpu.SemaphoreType.DMA,
    ]
    if with_counts:
        scratch.append(pltpu.VMEM((1, N), jnp.float32))

    def body(*refs):
        if with_counts:
            (x_hbm, src_hbm, dst_hbm, z_hbm, p_hbm, cnt_hbm,
             sidx_all, didx_all, rows0, rows1, acc, g0, g1, cnt_local) = refs
        else:
            (x_hbm, src_hbm, dst_hbm, z_hbm, p_hbm,
             sidx_all, didx_all, rows0, rows1, acc, g0, g1) = refs

        cid = jax.lax.axis_index("c")
        sid = jax.lax.axis_index("s")
        wid = cid * NS + sid

        pltpu.sync_copy(z_hbm, rows0)
        for r in range(RPT // K):
            pltpu.async_copy(rows0, acc.at[pl.ds(sid * RPT + r * K, K)], g0)
        for r in range(RPT // K):
            pltpu.make_async_copy(z_hbm, rows1, g0).wait()

        if with_counts:
            zero16 = jnp.zeros((16,), jnp.float32)

            @pl.loop(0, N // 16)
            def _(i):
                cnt_local[0, pl.ds(i * 16, 16)] = zero16

            izero = jnp.zeros((16,), jnp.int32)
            fone = jnp.full((16,), 1.0, jnp.float32)

            def count_chunk(j):
                @pl.loop(0, K // 16)
                def _(m):
                    dvec = didx_all[j, pl.ds(m * 16, 16)]
                    plsc.addupdate_scatter(cnt_local, [izero, dvec], fone)
        else:
            def count_chunk(j):
                pass

        plsc.subcore_barrier()

        @pl.loop(0, NG)
        def _(g):
            pltpu.sync_copy(src_hbm.at[wid, g], sidx_all)
            pltpu.sync_copy(dst_hbm.at[wid, g], didx_all)
            pltpu.async_copy(x_hbm.at[sidx_all.at[0]], rows0, g0)

            @pl.loop(0, (GC - 1) // 2)
            def _(r):
                j0 = 2 * r
                pltpu.async_copy(x_hbm.at[sidx_all.at[j0 + 1]], rows1, g1)
                pltpu.make_async_copy(z_hbm, rows0, g0).wait()
                pltpu.sync_copy(rows0, acc.at[didx_all.at[j0]], add=True)
                pltpu.async_copy(x_hbm.at[sidx_all.at[j0 + 2]], rows0, g0)
                count_chunk(j0)
                pltpu.make_async_copy(z_hbm, rows1, g1).wait()
                pltpu.sync_copy(rows1, acc.at[didx_all.at[j0 + 1]], add=True)
                count_chunk(j0 + 1)

            pltpu.make_async_copy(z_hbm, rows0, g0).wait()
            pltpu.sync_copy(rows0, acc.at[didx_all.at[GC - 1]], add=True)
            count_chunk(GC - 1)

        plsc.subcore_barrier()

        bufs = (rows0, rows1)
        sems = (g0, g1)
        for r in range(RPT // K):
            b = r % 2
            rbase = sid * RPT + r * K
            if r >= 2:
                pltpu.make_async_copy(z_hbm, bufs[b], sems[b]).wait()
            pltpu.sync_copy(acc.at[pl.ds(rbase, K)], bufs[b])
            pltpu.async_copy(bufs[b], p_hbm.at[cid, pl.ds(rbase, K)], sems[b])
        pltpu.make_async_copy(z_hbm, rows0, g0).wait()
        pltpu.make_async_copy(z_hbm, rows1, g1).wait()

        if with_counts:
            pltpu.sync_copy(cnt_local, cnt_hbm.at[wid])

    cp = pltpu.CompilerParams()
    if with_counts and "needs_layout_passes" in pltpu.CompilerParams.__dataclass_fields__:
        cp = dataclasses.replace(cp, needs_layout_passes=False)
    return functools.partial(
        pl.kernel, out_type=out_type, mesh=_mesh, scratch_types=scratch,
        compiler_params=cp)(body)


_sc_agg_counts = _make_sc_agg(True)
_sc_agg = _make_sc_agg(False)


BN = 1000


def _tc_body(p_ref, cnt_ref, x_ref, wrel_ref, b_ref, wroot_ref, o_ref):
    psum = p_ref[0] + p_ref[1]
    c = jnp.sum(cnt_ref[...], axis=1, keepdims=True)
    agg = psum * (1.0 / jnp.maximum(c, 1.0))
    h = jax.lax.dot_general(agg, wrel_ref[...],
                            (((1,), (1,)), ((), ())),
                            preferred_element_type=jnp.float32)
    h = h + jax.lax.dot_general(x_ref[...], wroot_ref[...],
                                (((1,), (1,)), ((), ())),
                                preferred_element_type=jnp.float32)
    h = h + b_ref[...]
    o_ref[...] = jnp.where(h > 0.0, h, jnp.exp(h) - 1.0)


def _tc_epilogue(p, cnt, x, w_rel, b, w_root):
    return pl.pallas_call(
        _tc_body,
        grid=(N // BN,),
        in_specs=[
            pl.BlockSpec((NC, BN, D), lambda i: (0, i, 0)),
            pl.BlockSpec((BN, NW), lambda i: (i, 0)),
            pl.BlockSpec((BN, D), lambda i: (i, 0)),
            pl.BlockSpec((H, D), lambda i: (0, 0)),
            pl.BlockSpec((1, H), lambda i: (0, 0)),
            pl.BlockSpec((H, D), lambda i: (0, 0)),
        ],
        out_specs=pl.BlockSpec((BN, H), lambda i: (i, 0)),
        out_shape=jax.ShapeDtypeStruct((N, H), jnp.float32),
    )(p, cnt, x, w_rel, b, w_root)


def kernel(node_embedding, edge_index, W_rel1, b1, W_root1, W_rel2, b2, W_root2):
    x = node_embedding
    src = edge_index[0]
    dst = edge_index[1]
    z = jnp.zeros((K, D), jnp.float32)

    src3 = src.reshape(NW, NG, GC, K)
    dst3 = dst.reshape(NW, NG, GC, K)

    p1, cnt32 = _sc_agg_counts(x, src3, dst3, z)
    cnt = cnt32.reshape(NW, N).T
    x1 = _tc_epilogue(p1, cnt, x, W_rel1, b1.reshape(1, H), W_root1)
    p2 = _sc_agg(x1, src3, dst3, z)
    x2 = _tc_epilogue(p2, cnt, x1, W_rel2, b2.reshape(1, H), W_root2)
    return x2
